# baseline (speedup 1.0000x reference)
import functools

import jax
import jax.numpy as jnp
from jax import lax
from jax.experimental import pallas as pl
from jax.experimental.pallas import tpu as pltpu
from jax.experimental.pallas import tpu_sc as plsc

N_FACTORS = 64
BATCH = 16384
NC = 2
NS = 16
NW = NC * NS
B_PER_W = BATCH // NW
N_CHUNKS = 4
CHUNK = B_PER_W // N_CHUNKS
GROUPS = B_PER_W // 16


def _sc_body(uidx_hbm, iidx_hbm, ufac_hbm, ifac_hbm, out_hbm,
             uidx_v, iidx_v, urows_v, vrows_v, out_v, sem):
    wid = lax.axis_index("s") * NC + lax.axis_index("c")
    base = wid * B_PER_W

    pltpu.sync_copy(uidx_hbm.at[wid], uidx_v)
    pltpu.sync_copy(iidx_hbm.at[wid], iidx_v)

    for j in range(N_CHUNKS):
        pltpu.async_copy(ufac_hbm.at[uidx_v.at[j]],
                         urows_v.at[pl.ds(j * CHUNK, CHUNK)], sem).wait()
        pltpu.async_copy(ifac_hbm.at[iidx_v.at[j]],
                         vrows_v.at[pl.ds(j * CHUNK, CHUNK)], sem).wait()

    lane = lax.iota(jnp.int32, 16)
    dnums = lax.GatherDimensionNumbers(
        offset_dims=(), collapsed_slice_dims=(0,), start_index_map=(0,))

    def rot(x, k):
        idx = (lane + k) % 16
        return lax.gather(x, idx[:, None], dnums, slice_sizes=(1,),
                          mode=lax.GatherScatterMode.PROMISE_IN_BOUNDS)

    def lane_sum(x):
        for k in (8, 4, 2, 1):
            x = x + rot(x, k)
        return x

    def group(g, carry):
        acc = jnp.zeros((16,), jnp.float32)
        for r in range(16):
            row = g * 16 + r
            p = jnp.zeros((16,), jnp.float32)
            for k in range(N_FACTORS // 16):
                u = urows_v[row, pl.ds(k * 16, 16)]
                v = vrows_v[row, pl.ds(k * 16, 16)]
                p = p + u * v
            s = lane_sum(p)
            acc = jnp.where(lane == r, s, acc)
        out_v[pl.ds(g * 16, 16)] = acc
        return carry

    lax.fori_loop(0, GROUPS, group, 0)

    pltpu.sync_copy(out_v, out_hbm.at[pl.ds(base, B_PER_W)])


@jax.jit
def _run(uidx, iidx, ufac, ifac):
    mesh = plsc.VectorSubcoreMesh(core_axis_name="c", subcore_axis_name="s")
    return pl.kernel(
        _sc_body,
        out_type=jax.ShapeDtypeStruct((BATCH,), jnp.float32),
        mesh=mesh,
        compiler_params=pltpu.CompilerParams(use_tc_tiling_on_sc=False),
        scratch_types=[
            pltpu.VMEM((N_CHUNKS, CHUNK), jnp.int32),
            pltpu.VMEM((N_CHUNKS, CHUNK), jnp.int32),
            pltpu.VMEM((B_PER_W, N_FACTORS), jnp.float32),
            pltpu.VMEM((B_PER_W, N_FACTORS), jnp.float32),
            pltpu.VMEM((B_PER_W,), jnp.float32),
            pltpu.SemaphoreType.DMA,
        ],
    )(uidx, iidx, ufac, ifac)


def kernel(user_idx, item_idx, user_factors, item_factors):
    uidx = user_idx.astype(jnp.int32).reshape(NW, N_CHUNKS, CHUNK)
    iidx = item_idx.astype(jnp.int32).reshape(NW, N_CHUNKS, CHUNK)
    return _run(uidx, iidx, user_factors, item_factors)
